# baseline (speedup 1.0000x reference)
import functools
import math

import jax
import jax.numpy as jnp
import numpy as np
from jax import lax
from jax.experimental import pallas as pl
from jax.experimental.pallas import tpu as pltpu
from jax.experimental.pallas import tpu_sc as plsc

_B = 16384
_NF = 26
_ED = 32
_B26 = _B * _NF
_NC = 2
_NS = 16
_NW = _NC * _NS
_PER_W = _B26 // _NW
_CHUNK = 128
_NBUF = 4
_NSTEP = _PER_W // (_CHUNK * _NBUF)
_OFFS = np.arange(_NF, dtype=np.int32) * 100000

_SC_PARAMS = pltpu.CompilerParams(use_tc_tiling_on_sc=False,
                                  needs_layout_passes=False)
_SC_MESH = dict(core_axis_name="c", subcore_axis_name="s")


@functools.partial(
    pl.kernel,
    out_type=jax.ShapeDtypeStruct((_B26, _ED), jnp.float32),
    mesh=plsc.VectorSubcoreMesh(**_SC_MESH),
    scratch_types=[
        pltpu.VMEM((_NBUF, _CHUNK), jnp.int32),
        pltpu.VMEM((_NBUF, _CHUNK, _ED), jnp.float32),
        pltpu.SemaphoreType.DMA((_NBUF,)),
    ],
    compiler_params=_SC_PARAMS,
)
def _sc_gather_emb(idx_hbm, etab, emb_out, idx_v, emb_v, sems):
    wid = lax.axis_index("s") * _NC + lax.axis_index("c")
    base = wid * _PER_W

    @pl.loop(0, _NSTEP)
    def step(g):
        pos0 = base + g * (_CHUNK * _NBUF)
        copies = []
        for b in range(_NBUF):
            pos = pos0 + b * _CHUNK
            pltpu.sync_copy(idx_hbm.at[pl.ds(pos, _CHUNK)], idx_v.at[b])
            copies.append(
                pltpu.async_copy(etab.at[idx_v.at[b]], emb_v.at[b], sems.at[b]))
        for b in range(_NBUF):
            pos = pos0 + b * _CHUNK
            copies[b].wait()
            pltpu.sync_copy(emb_v.at[b], emb_out.at[pl.ds(pos, _CHUNK)])


@functools.partial(
    pl.kernel,
    out_type=jax.ShapeDtypeStruct((_B26,), jnp.float32),
    mesh=plsc.VectorSubcoreMesh(**_SC_MESH),
    scratch_types=[
        pltpu.VMEM((_NBUF, _CHUNK), jnp.int32),
        pltpu.VMEM((_NBUF, _CHUNK), jnp.int32),
        pltpu.VMEM((_NBUF, _CHUNK, 16), jnp.float32),
        pltpu.VMEM((_NBUF, _CHUNK), jnp.float32),
        pltpu.SemaphoreType.DMA((_NBUF,)),
    ],
    compiler_params=_SC_PARAMS,
)
def _sc_gather_lin(idx_hbm, ltab16, lin_out,
                   idx16_v, lane_v, lrow_v, lin_v, sems):
    wid = lax.axis_index("s") * _NC + lax.axis_index("c")
    base = wid * _PER_W
    iota = lax.iota(jnp.int32, 16)

    @pl.loop(0, _NSTEP)
    def step(g):
        pos0 = base + g * (_CHUNK * _NBUF)
        copies = []
        for b in range(_NBUF):
            pos = pos0 + b * _CHUNK
            pltpu.sync_copy(idx_hbm.at[pl.ds(pos, _CHUNK)], idx16_v.at[b])
            for v in range(_CHUNK // 16):
                sl = pl.ds(v * 16, 16)
                ivec = idx16_v[b, sl]
                lane_v[b, sl] = lax.bitwise_and(ivec, 15)
                idx16_v[b, sl] = lax.shift_right_logical(ivec, 4)
            copies.append(
                pltpu.async_copy(ltab16.at[idx16_v.at[b]], lrow_v.at[b],
                                 sems.at[b]))
        for b in range(_NBUF):
            pos = pos0 + b * _CHUNK
            copies[b].wait()
            for v in range(_CHUNK // 16):
                sl = pl.ds(v * 16, 16)
                rows = iota + v * 16
                lin_v[b, sl] = plsc.load_gather(lrow_v.at[b],
                                                [rows, lane_v[b, sl]])
            pltpu.sync_copy(lin_v.at[b], lin_out.at[pl.ds(pos, _CHUNK)])


def _mlp_body(h_ref, lv_ref, w0, b0, w1, b1, w2, b2, wo, bo, out_ref):
    f32 = jnp.float32
    h = h_ref[...]
    h = jnp.maximum(jnp.dot(h, w0[...], preferred_element_type=f32) + b0[...], 0.0)
    h = jnp.maximum(jnp.dot(h, w1[...], preferred_element_type=f32) + b1[...], 0.0)
    h = jnp.maximum(jnp.dot(h, w2[...], preferred_element_type=f32) + b2[...], 0.0)
    o = jnp.maximum(jnp.dot(h, wo[...], preferred_element_type=f32) + bo[...], 0.0)
    out_ref[...] = o + jnp.sum(lv_ref[...], axis=1, keepdims=True)


def _mlp(h, lv, w0, b0, w1, b1, w2, b2, wo, bo, bc=2048):
    grid = (_B // bc,)
    full = lambda a: pl.BlockSpec(a.shape, lambda i: (0,) * a.ndim)
    return pl.pallas_call(
        _mlp_body,
        grid=grid,
        in_specs=[
            pl.BlockSpec((bc, h.shape[1]), lambda i: (i, 0)),
            pl.BlockSpec((bc, _NF), lambda i: (i, 0)),
            full(w0), full(b0), full(w1), full(b1),
            full(w2), full(b2), full(wo), full(bo),
        ],
        out_specs=pl.BlockSpec((bc, 1), lambda i: (i, 0)),
        out_shape=jax.ShapeDtypeStruct((_B, 1), jnp.float32),
    )(h, lv, w0, b0, w1, b1, w2, b2, wo, bo)


def kernel(x, embed_table, lin_table, lin_bias, W0, b0, g0, beta0,
           W1, b1, g1, beta1, W2, b2, g2, beta2, Wout, bout):
    idx = (x + _OFFS[None, :]).reshape(-1)
    ltab16 = lin_table.reshape(-1, 16)
    idx, ltab16 = lax.optimization_barrier((idx, ltab16))
    lin_flat = _sc_gather_lin(idx, ltab16)
    emb_flat = _sc_gather_emb(idx, embed_table)
    h = emb_flat.reshape(_B, _NF * _ED)
    lv = lin_flat.reshape(_B, _NF)

    c = 1.0 / math.sqrt(1.0 + 1e-5)
    s0, s1, s2 = g0 * c, g1 * c, g2 * c
    w0 = W0 * s0[None, :]
    w1 = W1 * s1[None, :]
    w2 = W2 * s2[None, :]
    bb0 = (b0 * s0 + beta0).reshape(1, -1)
    bb1 = (b1 * s1 + beta1).reshape(1, -1)
    bb2 = (b2 * s2 + beta2).reshape(1, -1)
    bo = (bout + lin_bias).reshape(1, 1)
    return _mlp(h, lv, w0, bb0, w1, bb1, w2, bb2, Wout, bo)


# speedup vs baseline: 1.0178x; 1.0178x over previous
import functools
import math

import jax
import jax.numpy as jnp
import numpy as np
from jax import lax
from jax.experimental import pallas as pl
from jax.experimental.pallas import tpu as pltpu
from jax.experimental.pallas import tpu_sc as plsc

_B = 16384
_NF = 26
_ED = 32
_B26 = _B * _NF
_NC = 2
_NS = 16
_NW = _NC * _NS
_PER_W = _B26 // _NW
_CHUNK = 128
_NBUF = 4
_NSTEP = _PER_W // (_CHUNK * _NBUF)
_OFFS = np.arange(_NF, dtype=np.int32) * 100000

_SC_PARAMS = pltpu.CompilerParams(use_tc_tiling_on_sc=False,
                                  needs_layout_passes=False)
_SC_MESH = dict(core_axis_name="c", subcore_axis_name="s")


@functools.partial(
    pl.kernel,
    out_type=jax.ShapeDtypeStruct((_B26, _ED), jnp.float32),
    mesh=plsc.VectorSubcoreMesh(**_SC_MESH),
    scratch_types=[
        pltpu.VMEM((_NBUF, _CHUNK), jnp.int32),
        pltpu.VMEM((_NBUF, _CHUNK, _ED), jnp.float32),
        pltpu.SemaphoreType.DMA((_NBUF,)),
    ],
    compiler_params=_SC_PARAMS,
)
def _sc_gather_emb(idx_hbm, etab, emb_out, idx_v, emb_v, sems):
    wid = lax.axis_index("s") * _NC + lax.axis_index("c")
    base = wid * _PER_W

    @pl.loop(0, _NSTEP)
    def step(g):
        pos0 = base + g * (_CHUNK * _NBUF)
        copies = []
        for b in range(_NBUF):
            pos = pos0 + b * _CHUNK
            pltpu.sync_copy(idx_hbm.at[pl.ds(pos, _CHUNK)], idx_v.at[b])
            copies.append(
                pltpu.async_copy(etab.at[idx_v.at[b]], emb_v.at[b], sems.at[b]))
        for b in range(_NBUF):
            pos = pos0 + b * _CHUNK
            copies[b].wait()
            pltpu.sync_copy(emb_v.at[b], emb_out.at[pl.ds(pos, _CHUNK)])


@functools.partial(
    pl.kernel,
    out_type=jax.ShapeDtypeStruct((_B26,), jnp.float32),
    mesh=plsc.VectorSubcoreMesh(**_SC_MESH),
    scratch_types=[
        pltpu.VMEM((_NBUF, _CHUNK), jnp.int32),
        pltpu.VMEM((_NBUF, _CHUNK), jnp.int32),
        pltpu.VMEM((_NBUF, _CHUNK, 16), jnp.float32),
        pltpu.VMEM((_NBUF, _CHUNK), jnp.float32),
        pltpu.SemaphoreType.DMA((_NBUF,)),
    ],
    compiler_params=_SC_PARAMS,
)
def _sc_gather_lin(idx_hbm, ltab16, lin_out,
                   idx16_v, lane_v, lrow_v, lin_v, sems):
    wid = lax.axis_index("s") * _NC + lax.axis_index("c")
    base = wid * _PER_W
    iota = lax.iota(jnp.int32, 16)

    @pl.loop(0, _NSTEP)
    def step(g):
        pos0 = base + g * (_CHUNK * _NBUF)
        copies = []
        for b in range(_NBUF):
            pos = pos0 + b * _CHUNK
            pltpu.sync_copy(idx_hbm.at[pl.ds(pos, _CHUNK)], idx16_v.at[b])
            for v in range(_CHUNK // 16):
                sl = pl.ds(v * 16, 16)
                ivec = idx16_v[b, sl]
                lane_v[b, sl] = lax.bitwise_and(ivec, 15)
                idx16_v[b, sl] = lax.shift_right_logical(ivec, 4)
            copies.append(
                pltpu.async_copy(ltab16.at[idx16_v.at[b]], lrow_v.at[b],
                                 sems.at[b]))
        for b in range(_NBUF):
            pos = pos0 + b * _CHUNK
            copies[b].wait()
            for v in range(_CHUNK // 16):
                sl = pl.ds(v * 16, 16)
                rows = iota + v * 16
                lin_v[b, sl] = plsc.load_gather(lrow_v.at[b],
                                                [rows, lane_v[b, sl]])
            pltpu.sync_copy(lin_v.at[b], lin_out.at[pl.ds(pos, _CHUNK)])


def _mlp_body(h_ref, lv_ref, w0, b0, w1, b1, w2, b2, wo, bo, out_ref):
    f32 = jnp.float32
    h = h_ref[...]
    h = jnp.maximum(jnp.dot(h, w0[...], preferred_element_type=f32) + b0[...], 0.0)
    h = jnp.maximum(jnp.dot(h, w1[...], preferred_element_type=f32) + b1[...], 0.0)
    h = jnp.maximum(jnp.dot(h, w2[...], preferred_element_type=f32) + b2[...], 0.0)
    o = jnp.maximum(jnp.dot(h, wo[...], preferred_element_type=f32) + bo[...], 0.0)
    out_ref[...] = o + jnp.sum(lv_ref[...], axis=1, keepdims=True)


def _mlp(h, lv, w0, b0, w1, b1, w2, b2, wo, bo, bc=2048):
    grid = (_B // bc,)
    full = lambda a: pl.BlockSpec(a.shape, lambda i: (0,) * a.ndim)
    return pl.pallas_call(
        _mlp_body,
        grid=grid,
        in_specs=[
            pl.BlockSpec((bc, h.shape[1]), lambda i: (i, 0)),
            pl.BlockSpec((bc, _NF), lambda i: (i, 0)),
            full(w0), full(b0), full(w1), full(b1),
            full(w2), full(b2), full(wo), full(bo),
        ],
        out_specs=pl.BlockSpec((bc, 1), lambda i: (i, 0)),
        out_shape=jax.ShapeDtypeStruct((_B, 1), jnp.float32),
    )(h, lv, w0, b0, w1, b1, w2, b2, wo, bo)


def kernel(x, embed_table, lin_table, lin_bias, W0, b0, g0, beta0,
           W1, b1, g1, beta1, W2, b2, g2, beta2, Wout, bout):
    idx = (x + _OFFS[None, :]).reshape(-1)
    ltab16 = lin_table.reshape(-1, 16)
    lin_flat = _sc_gather_lin(idx, ltab16)
    emb_flat = _sc_gather_emb(idx, embed_table)
    h = emb_flat.reshape(_B, _NF * _ED)
    lv = lin_flat.reshape(_B, _NF)

    c = 1.0 / math.sqrt(1.0 + 1e-5)
    s0, s1, s2 = g0 * c, g1 * c, g2 * c
    w0 = W0 * s0[None, :]
    w1 = W1 * s1[None, :]
    w2 = W2 * s2[None, :]
    bb0 = (b0 * s0 + beta0).reshape(1, -1)
    bb1 = (b1 * s1 + beta1).reshape(1, -1)
    bb2 = (b2 * s2 + beta2).reshape(1, -1)
    bo = (bout + lin_bias).reshape(1, 1)
    return _mlp(h, lv, w0, bb0, w1, bb1, w2, bb2, Wout, bo)
